# Initial kernel scaffold; baseline (speedup 1.0000x reference)
import functools

import numpy as np
import jax
import jax.numpy as jnp
from jax import lax
from jax.experimental import pallas as pl
from jax.experimental.pallas import tpu as pltpu
from jax.experimental.pallas import tpu_sc as plsc

_N = 10000
_E = 320000
_D = 128
_C = 64
_LAMDA = 0.5
_ALPHA = 0.1
_TH1 = float(np.log(_LAMDA / 2.0 + 1.0))
_TH2 = float(np.log(_LAMDA / 3.0 + 1.0))

_NC = 2
_NS = 16
_NW = _NC * _NS
_EPW = _E // _NW
_K = 80
_NCHUNK = _EPW // _K
_RPT = _N // _NS
_ZROWS = 125

_BLK = 500


_sc_mesh = plsc.VectorSubcoreMesh(core_axis_name="c", subcore_axis_name="s")


@functools.partial(
    pl.kernel,
    mesh=_sc_mesh,
    out_type=jax.ShapeDtypeStruct((_NC, _N, _D), jnp.float32),
    scratch_types=[
        pltpu.VMEM((_EPW,), jnp.int32),
        pltpu.VMEM((_NCHUNK, _K), jnp.int32),
        pltpu.VMEM((_EPW,), jnp.float32),
        pltpu.VMEM((_K, _D), jnp.float32),
        pltpu.VMEM((_ZROWS, _D), jnp.float32),
        pltpu.VMEM_SHARED((_N, _D), jnp.float32),
        pltpu.SemaphoreType.DMA,
    ],
)
def _spmm(g_hbm, src_hbm, dst_hbm, w_hbm, out_hbm,
          src_v, dst_v, w_v, rows_v, z_v, acc, sem):
    cid = lax.axis_index("c")
    sid = lax.axis_index("s")
    wid = sid * _NC + cid
    ebase = wid * _EPW

    pltpu.sync_copy(src_hbm.at[pl.ds(ebase, _EPW)], src_v)
    pltpu.sync_copy(w_hbm.at[pl.ds(ebase, _EPW)], w_v)
    pltpu.sync_copy(dst_hbm.at[wid], dst_v)

    zero = jnp.zeros((16,), jnp.float32)

    def _zrow(r, carry):
        for c in range(_D // 16):
            z_v[r, pl.ds(c * 16, 16)] = zero
        return carry

    lax.fori_loop(0, _ZROWS, _zrow, 0)
    rbase = sid * _RPT
    for i in range(_RPT // _ZROWS):
        pltpu.sync_copy(z_v, acc.at[pl.ds(rbase + i * _ZROWS, _ZROWS)])
    plsc.subcore_barrier()

    def _chunk(j, carry):
        eoff = j * _K
        pltpu.async_copy(
            g_hbm.at[src_v.at[pl.ds(eoff, _K)]], rows_v, sem).wait()

        def _edge(e, c2):
            w = w_v[eoff + e]
            for c in range(_D // 16):
                sl = (e, pl.ds(c * 16, 16))
                rows_v[sl] = rows_v[sl] * w
            return c2

        lax.fori_loop(0, _K, _edge, 0)
        pltpu.sync_copy(rows_v, acc.at[dst_v.at[j]], add=True)
        return carry

    lax.fori_loop(0, _NCHUNK, _chunk, 0)
    plsc.subcore_barrier()

    pltpu.sync_copy(acc.at[pl.ds(rbase, _RPT)],
                    out_hbm.at[cid, pl.ds(rbase, _RPT)])


def _eye():
    i = lax.broadcasted_iota(jnp.int32, (_D, _D), 0)
    j = lax.broadcasted_iota(jnp.int32, (_D, _D), 1)
    return (i == j).astype(jnp.float32)


def _tc1_body(x_ref, win_ref, bin_ref, w0_ref, w1_ref,
              h0_ref, g1_ref, q2_ref):
    x = x_ref[...]
    h0 = jnp.maximum(
        jnp.dot(x, win_ref[...], preferred_element_type=jnp.float32)
        + bin_ref[...], 0.0)
    h0_ref[...] = h0
    eye = _eye()
    m1 = _TH1 * w0_ref[...] + (1.0 - _TH1) * eye
    m2 = _TH2 * w1_ref[...] + (1.0 - _TH2) * eye
    g1_ref[...] = jnp.dot(h0, m1, preferred_element_type=jnp.float32)
    q2_ref[...] = _ALPHA * jnp.dot(h0, m2, preferred_element_type=jnp.float32)


def _tc2_body(s_ref, g1_ref, h0_ref, w1_ref, h1_ref, g2_ref):
    s = s_ref[0] + s_ref[1]
    h1 = jnp.maximum(
        (1.0 - _ALPHA) * s + _ALPHA * g1_ref[...] + h0_ref[...], 0.0)
    h1_ref[...] = h1
    m2 = _TH2 * w1_ref[...] + (1.0 - _TH2) * _eye()
    g2_ref[...] = jnp.dot(h1, m2, preferred_element_type=jnp.float32)


def _tc3_body(s_ref, q2_ref, h1_ref, wout_ref, bout_ref, out_ref):
    h2 = jnp.maximum(
        (1.0 - _ALPHA) * (s_ref[0] + s_ref[1]) + q2_ref[...] + h1_ref[...],
        0.0)
    logits = (jnp.dot(h2, wout_ref[...], preferred_element_type=jnp.float32)
              + bout_ref[...])
    m = jnp.max(logits, axis=1, keepdims=True)
    ex = jnp.exp(logits - m)
    lse = jnp.log(jnp.sum(ex, axis=1, keepdims=True))
    out_ref[...] = logits - m - lse


_row_spec = pl.BlockSpec((_BLK, _D), lambda i: (i, 0))
_w_spec = pl.BlockSpec((_D, _D), lambda i: (0, 0))
_s_spec = pl.BlockSpec((_NC, _BLK, _D), lambda i: (0, i, 0))
_grid = (_N // _BLK,)


def _tc1(x, w_in, b_in, w0, w1):
    f = pl.pallas_call(
        _tc1_body,
        grid=_grid,
        in_specs=[_row_spec, _w_spec,
                  pl.BlockSpec((1, _D), lambda i: (0, 0)),
                  _w_spec, _w_spec],
        out_specs=[_row_spec, _row_spec, _row_spec],
        out_shape=[jax.ShapeDtypeStruct((_N, _D), jnp.float32)] * 3,
    )
    return f(x, w_in, b_in.reshape(1, _D), w0, w1)


def _tc2(s, g1, h0, w1):
    f = pl.pallas_call(
        _tc2_body,
        grid=_grid,
        in_specs=[_s_spec, _row_spec, _row_spec, _w_spec],
        out_specs=[_row_spec, _row_spec],
        out_shape=[jax.ShapeDtypeStruct((_N, _D), jnp.float32)] * 2,
    )
    return f(s, g1, h0, w1)


def _tc3(s, q2, h1, w_out, b_out):
    f = pl.pallas_call(
        _tc3_body,
        grid=_grid,
        in_specs=[_s_spec, _row_spec, _row_spec,
                  pl.BlockSpec((_D, _C), lambda i: (0, 0)),
                  pl.BlockSpec((1, _C), lambda i: (0, 0))],
        out_specs=pl.BlockSpec((_BLK, _C), lambda i: (i, 0)),
        out_shape=jax.ShapeDtypeStruct((_N, _C), jnp.float32),
    )
    return f(s, q2, h1, w_out, b_out.reshape(1, _C))


def kernel(x, edge_index, edge_weight, W_in, b_in, W0, W1, W_out, b_out):
    src = edge_index[1].astype(jnp.int32)
    dst = edge_index[0].astype(jnp.int32).reshape(_NW, _NCHUNK, _K)
    w = edge_weight.astype(jnp.float32)

    h0, g1, q2 = _tc1(x, W_in, b_in, W0, W1)
    s1 = _spmm(g1, src, dst, w)
    h1, g2 = _tc2(s1, g1, h0, W1)
    s2 = _spmm(g2, src, dst, w)
    return _tc3(s2, q2, h1, W_out, b_out)


# speedup vs baseline: 6.1471x; 6.1471x over previous
import functools

import numpy as np
import jax
import jax.numpy as jnp
from jax import lax
from jax.experimental import pallas as pl
from jax.experimental.pallas import tpu as pltpu
from jax.experimental.pallas import tpu_sc as plsc

_N = 10000
_E = 320000
_D = 128
_C = 64
_LAMDA = 0.5
_ALPHA = 0.1
_TH1 = float(np.log(_LAMDA / 2.0 + 1.0))
_TH2 = float(np.log(_LAMDA / 3.0 + 1.0))

_NC = 2
_NS = 16
_NW = _NC * _NS
_EPW = _E // _NW
_K = 80
_NCHUNK = _EPW // _K
_NPAD = 10240
_RPT = _NPAD // _NS
_ZROWS = 128

_BLK = 1000


_sc_mesh = plsc.VectorSubcoreMesh(core_axis_name="c", subcore_axis_name="s")


@functools.partial(
    pl.kernel,
    mesh=_sc_mesh,
    out_type=jax.ShapeDtypeStruct((_NC, _NPAD, _D), jnp.float32),
    scratch_types=[
        pltpu.VMEM((_EPW,), jnp.int32),
        pltpu.VMEM((_NCHUNK, _K), jnp.int32),
        pltpu.VMEM((_EPW,), jnp.float32),
        pltpu.VMEM((_K, _D), jnp.float32),
        pltpu.VMEM_SHARED((_NPAD, _D), jnp.float32),
        pltpu.SemaphoreType.DMA,
    ],
)
def _spmm(g_hbm, src_hbm, dst_hbm, w_hbm, out_hbm,
          src_v, dst_v, w_v, rows_v, acc, sem):
    cid = lax.axis_index("c")
    sid = lax.axis_index("s")
    wid = sid * _NC + cid
    ebase = wid * _EPW

    pltpu.sync_copy(src_hbm.at[pl.ds(ebase, _EPW)], src_v)
    pltpu.sync_copy(w_hbm.at[pl.ds(ebase, _EPW)], w_v)
    pltpu.sync_copy(dst_hbm.at[wid], dst_v)

    zero = jnp.zeros((16,), jnp.float32)

    def _zrow(r, carry):
        for c in range(_D // 16):
            rows_v[r, pl.ds(c * 16, 16)] = zero
        return carry

    lax.fori_loop(0, _K, _zrow, 0)
    rbase = sid * _RPT
    for i in range(_RPT // _K):
        pltpu.sync_copy(rows_v, acc.at[pl.ds(rbase + i * _K, _K)])
    plsc.subcore_barrier()

    def _chunk(j, carry):
        eoff = j * _K
        pltpu.async_copy(
            g_hbm.at[src_v.at[pl.ds(eoff, _K)]], rows_v, sem).wait()

        def _egroup(b, c2):
            wv = w_v[pl.ds(eoff + b * 16, 16)]
            for k in range(16):
                w = wv[k]
                e = b * 16 + k
                for c in range(_D // 16):
                    sl = (e, pl.ds(c * 16, 16))
                    rows_v[sl] = rows_v[sl] * w
            return c2

        lax.fori_loop(0, _K // 16, _egroup, 0)
        pltpu.sync_copy(rows_v, acc.at[dst_v.at[j]], add=True)
        return carry

    lax.fori_loop(0, _NCHUNK, _chunk, 0)
    plsc.subcore_barrier()

    pltpu.sync_copy(acc.at[pl.ds(rbase, _RPT)],
                    out_hbm.at[cid, pl.ds(rbase, _RPT)])


def _eye():
    i = lax.broadcasted_iota(jnp.int32, (_D, _D), 0)
    j = lax.broadcasted_iota(jnp.int32, (_D, _D), 1)
    return (i == j).astype(jnp.float32)


def _tc1_body(x_ref, win_ref, bin_ref, w0_ref, w1_ref,
              h0_ref, g1_ref, q2_ref):
    x = x_ref[...]
    h0 = jnp.maximum(
        jnp.dot(x, win_ref[...], preferred_element_type=jnp.float32)
        + bin_ref[...], 0.0)
    h0_ref[...] = h0
    eye = _eye()
    m1 = _TH1 * w0_ref[...] + (1.0 - _TH1) * eye
    m2 = _TH2 * w1_ref[...] + (1.0 - _TH2) * eye
    g1_ref[...] = jnp.dot(h0, m1, preferred_element_type=jnp.float32)
    q2_ref[...] = _ALPHA * jnp.dot(h0, m2, preferred_element_type=jnp.float32)


def _tc2_body(s_ref, g1_ref, h0_ref, w1_ref, h1_ref, g2_ref):
    s = s_ref[0] + s_ref[1]
    h1 = jnp.maximum(
        (1.0 - _ALPHA) * s + _ALPHA * g1_ref[...] + h0_ref[...], 0.0)
    h1_ref[...] = h1
    m2 = _TH2 * w1_ref[...] + (1.0 - _TH2) * _eye()
    g2_ref[...] = jnp.dot(h1, m2, preferred_element_type=jnp.float32)


def _tc3_body(s_ref, q2_ref, h1_ref, wout_ref, bout_ref, out_ref):
    h2 = jnp.maximum(
        (1.0 - _ALPHA) * (s_ref[0] + s_ref[1]) + q2_ref[...] + h1_ref[...],
        0.0)
    logits = (jnp.dot(h2, wout_ref[...], preferred_element_type=jnp.float32)
              + bout_ref[...])
    m = jnp.max(logits, axis=1, keepdims=True)
    ex = jnp.exp(logits - m)
    lse = jnp.log(jnp.sum(ex, axis=1, keepdims=True))
    out_ref[...] = logits - m - lse


_row_spec = pl.BlockSpec((_BLK, _D), lambda i: (i, 0))
_w_spec = pl.BlockSpec((_D, _D), lambda i: (0, 0))
_s_spec = pl.BlockSpec((_NC, _BLK, _D), lambda i: (0, i, 0))
_grid = (_N // _BLK,)


def _tc1(x, w_in, b_in, w0, w1):
    f = pl.pallas_call(
        _tc1_body,
        grid=_grid,
        in_specs=[_row_spec, _w_spec,
                  pl.BlockSpec((1, _D), lambda i: (0, 0)),
                  _w_spec, _w_spec],
        out_specs=[_row_spec, _row_spec, _row_spec],
        out_shape=[jax.ShapeDtypeStruct((_N, _D), jnp.float32)] * 3,
    )
    return f(x, w_in, b_in.reshape(1, _D), w0, w1)


def _tc2(s, g1, h0, w1):
    f = pl.pallas_call(
        _tc2_body,
        grid=_grid,
        in_specs=[_s_spec, _row_spec, _row_spec, _w_spec],
        out_specs=[_row_spec, _row_spec],
        out_shape=[jax.ShapeDtypeStruct((_N, _D), jnp.float32)] * 2,
    )
    return f(s, g1, h0, w1)


def _tc3(s, q2, h1, w_out, b_out):
    f = pl.pallas_call(
        _tc3_body,
        grid=_grid,
        in_specs=[_s_spec, _row_spec, _row_spec,
                  pl.BlockSpec((_D, _C), lambda i: (0, 0)),
                  pl.BlockSpec((1, _C), lambda i: (0, 0))],
        out_specs=pl.BlockSpec((_BLK, _C), lambda i: (i, 0)),
        out_shape=jax.ShapeDtypeStruct((_N, _C), jnp.float32),
    )
    return f(s, q2, h1, w_out, b_out.reshape(1, _C))


def kernel(x, edge_index, edge_weight, W_in, b_in, W0, W1, W_out, b_out):
    src = edge_index[1].astype(jnp.int32)
    dst = edge_index[0].astype(jnp.int32).reshape(_NW, _NCHUNK, _K)
    w = edge_weight.astype(jnp.float32)

    h0, g1, q2 = _tc1(x, W_in, b_in, W0, W1)
    s1 = _spmm(g1, src, dst, w)[:, :_N, :]
    h1, g2 = _tc2(s1, g1, h0, W1)
    s2 = _spmm(g2, src, dst, w)[:, :_N, :]
    return _tc3(s2, q2, h1, W_out, b_out)


# speedup vs baseline: 6.9706x; 1.1340x over previous
import functools

import numpy as np
import jax
import jax.numpy as jnp
from jax import lax
from jax.experimental import pallas as pl
from jax.experimental.pallas import tpu as pltpu
from jax.experimental.pallas import tpu_sc as plsc

_N = 10000
_E = 320000
_D = 128
_C = 64
_LAMDA = 0.5
_ALPHA = 0.1
_TH1 = float(np.log(_LAMDA / 2.0 + 1.0))
_TH2 = float(np.log(_LAMDA / 3.0 + 1.0))

_NC = 2
_NS = 16
_NW = _NC * _NS
_EPW = _E // _NW
_K = 80
_NCHUNK = 126
_EPP = _K * _NCHUNK
_NPAD = 10240
_RPT = _NPAD // _NS

_BLK = 1000


_sc_mesh = plsc.VectorSubcoreMesh(core_axis_name="c", subcore_axis_name="s")


@functools.partial(
    pl.kernel,
    mesh=_sc_mesh,
    out_type=jax.ShapeDtypeStruct((_NC, _NPAD, _D), jnp.float32),
    scratch_types=[
        pltpu.VMEM((_NCHUNK, _K), jnp.int32),
        pltpu.VMEM((3, _K), jnp.int32),
        pltpu.VMEM((3, _K), jnp.float32),
        pltpu.VMEM((3, _K, _D), jnp.float32),
        pltpu.VMEM_SHARED((_NPAD, _D), jnp.float32),
        pltpu.SemaphoreType.DMA, pltpu.SemaphoreType.DMA,
        pltpu.SemaphoreType.DMA, pltpu.SemaphoreType.DMA,
        pltpu.SemaphoreType.DMA, pltpu.SemaphoreType.DMA,
        pltpu.SemaphoreType.DMA, pltpu.SemaphoreType.DMA,
        pltpu.SemaphoreType.DMA, pltpu.SemaphoreType.DMA,
        pltpu.SemaphoreType.DMA, pltpu.SemaphoreType.DMA,
    ],
)
def _spmm(g_hbm, src_hbm, dst_hbm, w_hbm, out_hbm,
          dst_v, src_v, w_v, rows_v, acc,
          g0, g1, g2, s0, s1, s2, i0, i1, i2, w0, w1, w2):
    gsem = (g0, g1, g2)
    ssem = (s0, s1, s2)
    isem = (i0, i1, i2)
    wsem = (w0, w1, w2)
    cid = lax.axis_index("c")
    sid = lax.axis_index("s")
    wid = sid * _NC + cid
    ebase = wid * _EPP

    pltpu.sync_copy(dst_hbm.at[wid], dst_v)
    for c0 in range(2):
        pltpu.sync_copy(src_hbm.at[pl.ds(ebase + c0 * _K, _K)],
                        src_v.at[c0])
        pltpu.sync_copy(w_hbm.at[pl.ds(ebase + c0 * _K, _K)],
                        w_v.at[c0])

    zero = jnp.zeros((16,), jnp.float32)

    def _zrow(r, carry):
        for c in range(_D // 16):
            rows_v[0, r, pl.ds(c * 16, 16)] = zero
        return carry

    lax.fori_loop(0, _K, _zrow, 0)
    rbase = sid * _RPT
    for i in range(_RPT // _K):
        pltpu.sync_copy(rows_v.at[0], acc.at[pl.ds(rbase + i * _K, _K)])
    _rem = _RPT - (_RPT // _K) * _K
    if _rem:
        pltpu.sync_copy(rows_v.at[0, pl.ds(0, _rem)],
                        acc.at[pl.ds(rbase + (_RPT // _K) * _K, _rem)])
    plsc.subcore_barrier()

    pltpu.async_copy(g_hbm.at[src_v.at[0]], rows_v.at[0], gsem[0])

    def _outer(it, carry):
        j0 = it * 3
        for r in range(3):
            j = j0 + r
            s, sn, sp = r, (r + 1) % 3, (r + 2) % 3

            @pl.when(j >= 2)
            def _(j=j, sn=sn):
                pltpu.make_async_copy(
                    rows_v.at[sn], acc.at[dst_v.at[j - 2]], ssem[sn]).wait()

            @pl.when(j + 1 < _NCHUNK)
            def _(j=j, sn=sn):
                @pl.when(j >= 1)
                def _(j=j, sn=sn):
                    pltpu.make_async_copy(
                        src_hbm.at[pl.ds(ebase + (j + 1) * _K, _K)],
                        src_v.at[sn], isem[sn]).wait()
                pltpu.async_copy(
                    g_hbm.at[src_v.at[sn]], rows_v.at[sn], gsem[sn])

            @pl.when(j + 2 < _NCHUNK)
            def _(j=j, sp=sp):
                off = ebase + (j + 2) * _K
                pltpu.async_copy(src_hbm.at[pl.ds(off, _K)],
                                 src_v.at[sp], isem[sp])
                pltpu.async_copy(w_hbm.at[pl.ds(off, _K)],
                                 w_v.at[sp], wsem[sp])

            pltpu.make_async_copy(
                g_hbm.at[src_v.at[s]], rows_v.at[s], gsem[s]).wait()

            @pl.when(j >= 2)
            def _(j=j, s=s):
                pltpu.make_async_copy(
                    w_hbm.at[pl.ds(ebase + j * _K, _K)],
                    w_v.at[s], wsem[s]).wait()

            def _egroup(b, c2, s=s):
                wv = w_v[s, pl.ds(b * 16, 16)]
                for k in range(16):
                    w = wv[k]
                    e = b * 16 + k
                    for c in range(_D // 16):
                        sl = (s, e, pl.ds(c * 16, 16))
                        rows_v[sl] = rows_v[sl] * w
                return c2

            lax.fori_loop(0, _K // 16, _egroup, 0)

            pltpu.async_copy(
                rows_v.at[s], acc.at[dst_v.at[j]], ssem[s], add=True)
        return carry

    lax.fori_loop(0, _NCHUNK // 3, _outer, 0)

    pltpu.make_async_copy(
        rows_v.at[(_NCHUNK - 2) % 3], acc.at[dst_v.at[_NCHUNK - 2]],
        ssem[(_NCHUNK - 2) % 3]).wait()
    pltpu.make_async_copy(
        rows_v.at[(_NCHUNK - 1) % 3], acc.at[dst_v.at[_NCHUNK - 1]],
        ssem[(_NCHUNK - 1) % 3]).wait()
    plsc.subcore_barrier()

    pltpu.sync_copy(acc.at[pl.ds(rbase, _RPT)],
                    out_hbm.at[cid, pl.ds(rbase, _RPT)])


def _eye():
    i = lax.broadcasted_iota(jnp.int32, (_D, _D), 0)
    j = lax.broadcasted_iota(jnp.int32, (_D, _D), 1)
    return (i == j).astype(jnp.float32)


def _tc1_body(x_ref, win_ref, bin_ref, w0_ref, w1_ref,
              h0_ref, g1_ref, q2_ref):
    x = x_ref[...]
    h0 = jnp.maximum(
        jnp.dot(x, win_ref[...], preferred_element_type=jnp.float32)
        + bin_ref[...], 0.0)
    h0_ref[...] = h0
    eye = _eye()
    m1 = _TH1 * w0_ref[...] + (1.0 - _TH1) * eye
    m2 = _TH2 * w1_ref[...] + (1.0 - _TH2) * eye
    g1_ref[...] = jnp.dot(h0, m1, preferred_element_type=jnp.float32)
    q2_ref[...] = _ALPHA * jnp.dot(h0, m2, preferred_element_type=jnp.float32)


def _tc2_body(s_ref, g1_ref, h0_ref, w1_ref, h1_ref, g2_ref):
    s = s_ref[0] + s_ref[1]
    h1 = jnp.maximum(
        (1.0 - _ALPHA) * s + _ALPHA * g1_ref[...] + h0_ref[...], 0.0)
    h1_ref[...] = h1
    m2 = _TH2 * w1_ref[...] + (1.0 - _TH2) * _eye()
    g2_ref[...] = jnp.dot(h1, m2, preferred_element_type=jnp.float32)


def _tc3_body(s_ref, q2_ref, h1_ref, wout_ref, bout_ref, out_ref):
    h2 = jnp.maximum(
        (1.0 - _ALPHA) * (s_ref[0] + s_ref[1]) + q2_ref[...] + h1_ref[...],
        0.0)
    logits = (jnp.dot(h2, wout_ref[...], preferred_element_type=jnp.float32)
              + bout_ref[...])
    m = jnp.max(logits, axis=1, keepdims=True)
    ex = jnp.exp(logits - m)
    lse = jnp.log(jnp.sum(ex, axis=1, keepdims=True))
    out_ref[...] = logits - m - lse


_row_spec = pl.BlockSpec((_BLK, _D), lambda i: (i, 0))
_w_spec = pl.BlockSpec((_D, _D), lambda i: (0, 0))
_s_spec = pl.BlockSpec((_NC, _BLK, _D), lambda i: (0, i, 0))
_grid = (_N // _BLK,)


def _tc1(x, w_in, b_in, w0, w1):
    f = pl.pallas_call(
        _tc1_body,
        grid=_grid,
        in_specs=[_row_spec, _w_spec,
                  pl.BlockSpec((1, _D), lambda i: (0, 0)),
                  _w_spec, _w_spec],
        out_specs=[_row_spec, _row_spec, _row_spec],
        out_shape=[jax.ShapeDtypeStruct((_N, _D), jnp.float32)] * 3,
    )
    return f(x, w_in, b_in.reshape(1, _D), w0, w1)


def _tc2(s, g1, h0, w1):
    f = pl.pallas_call(
        _tc2_body,
        grid=_grid,
        in_specs=[_s_spec, _row_spec, _row_spec, _w_spec],
        out_specs=[_row_spec, _row_spec],
        out_shape=[jax.ShapeDtypeStruct((_N, _D), jnp.float32)] * 2,
    )
    return f(s, g1, h0, w1)


def _tc3(s, q2, h1, w_out, b_out):
    f = pl.pallas_call(
        _tc3_body,
        grid=_grid,
        in_specs=[_s_spec, _row_spec, _row_spec,
                  pl.BlockSpec((_D, _C), lambda i: (0, 0)),
                  pl.BlockSpec((1, _C), lambda i: (0, 0))],
        out_specs=pl.BlockSpec((_BLK, _C), lambda i: (i, 0)),
        out_shape=jax.ShapeDtypeStruct((_N, _C), jnp.float32),
    )
    return f(s, q2, h1, w_out, b_out.reshape(1, _C))


def kernel(x, edge_index, edge_weight, W_in, b_in, W0, W1, W_out, b_out):
    pad = ((0, 0), (0, _EPP - _EPW))
    src = jnp.pad(edge_index[1].astype(jnp.int32).reshape(_NW, _EPW),
                  pad).reshape(-1)
    dst = jnp.pad(edge_index[0].astype(jnp.int32).reshape(_NW, _EPW),
                  pad).reshape(_NW, _NCHUNK, _K)
    w = jnp.pad(edge_weight.astype(jnp.float32).reshape(_NW, _EPW),
                pad).reshape(-1)

    h0, g1, q2 = _tc1(x, W_in, b_in, W0, W1)
    s1 = _spmm(g1, src, dst, w)[:, :_N, :]
    h1, g2 = _tc2(s1, g1, h0, W1)
    s2 = _spmm(g2, src, dst, w)[:, :_N, :]
    return _tc3(s2, q2, h1, W_out, b_out)
